# Initial kernel scaffold; baseline (speedup 1.0000x reference)
import jax
import jax.numpy as jnp
from jax.experimental import pallas as pl


def kernel(input_obs, invalid_mask):
    raise NotImplementedError("write your pallas kernel here")


# speedup vs baseline: 1.0947x; 1.0947x over previous
import functools

import numpy as np

import jax
import jax.numpy as jnp
from jax import lax
from jax.experimental import pallas as pl
from jax.experimental.pallas import tpu as pltpu
from jax.experimental.pallas import tpu_sc as plsc

_B = 16384
_NA = 100
_OBS_W = 302
_L = 16
_NC = 2
_NS = 16
_NW = _NC * _NS
_RW = _B // _NW
_C = 128
_TPAD = _B + 128
_PEN = np.float32(3.0e38)
_GB = (0, 16, 32, 48, 64, 80, 84)
_BIG_IDX = np.int32(1 << 20)


@functools.partial(
    pl.kernel,
    out_type=jax.ShapeDtypeStruct((_B * _NA,), jnp.float32),
    mesh=plsc.VectorSubcoreMesh(core_axis_name="c", subcore_axis_name="s"),
    compiler_params=pltpu.CompilerParams(needs_layout_passes=False),
    scratch_types=[
        pltpu.VMEM((_TPAD,), jnp.float32),
        pltpu.VMEM((_TPAD,), jnp.float32),
        pltpu.VMEM((_C * _OBS_W,), jnp.float32),
        pltpu.VMEM((_C * _NA,), jnp.float32),
        pltpu.VMEM((_C * _NA,), jnp.float32),
    ],
)
def _selector(obs_hbm, txp_hbm, typ_hbm, mask_hbm, out_hbm,
              tx_v, ty_v, obs_v, mask_v, out_v):
    wid = lax.axis_index("s") * _NC + lax.axis_index("c")
    base = wid * _RW
    pltpu.sync_copy(txp_hbm, tx_v)
    pltpu.sync_copy(typ_hbm, ty_v)

    lane = lax.iota(jnp.int32, _L)
    one = jnp.float32(1.0)
    zero = jnp.float32(0.0)

    def do_chunk(r0):
        pltpu.sync_copy(obs_hbm.at[pl.ds(r0 * _OBS_W, _C * _OBS_W)], obs_v)
        pltpu.sync_copy(mask_hbm.at[pl.ds(r0 * _NA, _C * _NA)], mask_v)

        def row(r, carry):
            start = (jnp.int32(_NA) * (r0 + r)) & jnp.int32(_B - 1)
            robs = r * _OBS_W
            rrow = r * _NA
            vmin = None
            varg = None
            for gb in _GB:
                ax = plsc.load_gather(obs_v, [robs + (lane * 3 + 3 * gb)])
                ay = plsc.load_gather(obs_v, [robs + (lane * 3 + (3 * gb + 1))])
                tx = tx_v[pl.ds(start + gb, _L)]
                ty = ty_v[pl.ds(start + gb, _L)]
                mg = mask_v[pl.ds(rrow + gb, _L)]
                dx = ax - tx
                dy = ay - ty
                d = dx * dx + dy * dy + mg * _PEN
                ids = lane + gb
                if vmin is None:
                    vmin, varg = d, ids
                else:
                    lt = d < vmin
                    varg = jnp.where(lt, ids, varg)
                    vmin = jnp.where(lt, d, vmin)
            m = jnp.min(vmin)
            idx = jnp.min(jnp.where(vmin == m, varg, _BIG_IDX))
            for gb in _GB:
                oh = jnp.where(lane + gb == idx, one, zero).astype(jnp.float32)
                out_v[pl.ds(rrow + gb, _L)] = oh
            return carry

        lax.fori_loop(0, _C, row, 0)
        pltpu.sync_copy(out_v, out_hbm.at[pl.ds(r0 * _NA, _C * _NA)])

    for k in range(_RW // _C):
        do_chunk(base + k * _C)


def kernel(input_obs, invalid_mask):
    tx = input_obs[:, _OBS_W - 2]
    ty = input_obs[:, _OBS_W - 1]
    txp = jnp.concatenate([tx, tx[: _TPAD - _B]])
    typ = jnp.concatenate([ty, ty[: _TPAD - _B]])
    out = _selector(input_obs.reshape(-1), txp, typ, invalid_mask.reshape(-1))
    return out.reshape(_B, _NA)


# speedup vs baseline: 1.2759x; 1.1656x over previous
import functools

import numpy as np

import jax
import jax.numpy as jnp
from jax import lax
from jax.experimental import pallas as pl
from jax.experimental.pallas import tpu as pltpu
from jax.experimental.pallas import tpu_sc as plsc

_B = 16384
_NA = 100
_OBS_W = 302
_L = 16
_NC = 2
_NS = 16
_NW = _NC * _NS
_RW = _B // _NW
_C = 64
_NCH = _RW // _C
_TPAD = _B + 128
_PEN = np.float32(3.0e38)
_GB = (0, 16, 32, 48, 64, 80, 84)
_BIG_IDX = np.int32(1 << 20)
_UNROLL = 2


@functools.partial(
    pl.kernel,
    out_type=jax.ShapeDtypeStruct((_B * _NA,), jnp.float32),
    mesh=plsc.VectorSubcoreMesh(core_axis_name="c", subcore_axis_name="s"),
    compiler_params=pltpu.CompilerParams(needs_layout_passes=False),
    scratch_types=[
        pltpu.VMEM((_TPAD,), jnp.float32),
        pltpu.VMEM((_TPAD,), jnp.float32),
        pltpu.VMEM((_C * _OBS_W,), jnp.float32),
        pltpu.VMEM((_C * _OBS_W,), jnp.float32),
        pltpu.VMEM((_C * _NA,), jnp.float32),
        pltpu.VMEM((_C * _NA,), jnp.float32),
        pltpu.VMEM((_C * _NA,), jnp.float32),
        pltpu.VMEM((_C * _NA,), jnp.float32),
        pltpu.SemaphoreType.DMA,
        pltpu.SemaphoreType.DMA,
        pltpu.SemaphoreType.DMA,
        pltpu.SemaphoreType.DMA,
    ],
)
def _selector(obs_hbm, txp_hbm, typ_hbm, mask_hbm, out_hbm,
              tx_v, ty_v, obs_v0, obs_v1, mask_v0, mask_v1,
              out_v0, out_v1, sem_in0, sem_in1, sem_out0, sem_out1):
    wid = lax.axis_index("s") * _NC + lax.axis_index("c")
    base = wid * _RW
    pltpu.sync_copy(txp_hbm, tx_v)
    pltpu.sync_copy(typ_hbm, ty_v)

    obs_v = (obs_v0, obs_v1)
    mask_v = (mask_v0, mask_v1)
    out_v = (out_v0, out_v1)
    sem_in = (sem_in0, sem_in1)
    sem_out = (sem_out0, sem_out1)

    lane = lax.iota(jnp.int32, _L)
    one = jnp.float32(1.0)
    zero = jnp.float32(0.0)

    def start_in(k, b):
        r0 = base + k * _C
        c1 = pltpu.async_copy(
            obs_hbm.at[pl.ds(r0 * _OBS_W, _C * _OBS_W)], obs_v[b], sem_in[b])
        c2 = pltpu.async_copy(
            mask_hbm.at[pl.ds(r0 * _NA, _C * _NA)], mask_v[b], sem_in[b])
        return (c1, c2)

    pending_in = start_in(0, 0)
    pending_out = [None, None]
    for k in range(_NCH):
        b = k & 1
        for c in pending_in:
            c.wait()
        if k + 1 < _NCH:
            pending_in = start_in(k + 1, 1 - b)
        if pending_out[b] is not None:
            pending_out[b].wait()
        r0 = base + k * _C
        ob, mb, tb = obs_v[b], mask_v[b], out_v[b]

        @plsc.parallel_loop(0, _C, unroll=_UNROLL)
        def row(r):
            start = (jnp.int32(_NA) * (r0 + r)) & jnp.int32(_B - 1)
            robs = r * _OBS_W
            rrow = r * _NA
            vmin = None
            varg = None
            for gb in _GB:
                ax = plsc.load_gather(ob, [robs + (lane * 3 + 3 * gb)])
                ay = plsc.load_gather(ob, [robs + (lane * 3 + (3 * gb + 1))])
                tx = tx_v[pl.ds(start + gb, _L)]
                ty = ty_v[pl.ds(start + gb, _L)]
                mg = mb[pl.ds(rrow + gb, _L)]
                dx = ax - tx
                dy = ay - ty
                d = dx * dx + dy * dy + mg * _PEN
                ids = lane + gb
                if vmin is None:
                    vmin, varg = d, ids
                else:
                    lt = d < vmin
                    varg = jnp.where(lt, ids, varg)
                    vmin = jnp.where(lt, d, vmin)
            m = jnp.min(vmin)
            idx = jnp.min(jnp.where(vmin == m, varg, _BIG_IDX))
            for gb in _GB:
                oh = jnp.where(lane + gb == idx, one, zero).astype(jnp.float32)
                tb[pl.ds(rrow + gb, _L)] = oh

        pending_out[b] = pltpu.async_copy(
            tb, out_hbm.at[pl.ds(r0 * _NA, _C * _NA)], sem_out[b])
    for b in (0, 1):
        if pending_out[b] is not None:
            pending_out[b].wait()


def kernel(input_obs, invalid_mask):
    tx = input_obs[:, _OBS_W - 2]
    ty = input_obs[:, _OBS_W - 1]
    txp = jnp.concatenate([tx, tx[: _TPAD - _B]])
    typ = jnp.concatenate([ty, ty[: _TPAD - _B]])
    out = _selector(input_obs.reshape(-1), txp, typ, invalid_mask.reshape(-1))
    return out.reshape(_B, _NA)


# speedup vs baseline: 1.8888x; 1.4803x over previous
import functools

import numpy as np

import jax
import jax.numpy as jnp
from jax import lax
from jax.experimental import pallas as pl
from jax.experimental.pallas import tpu as pltpu
from jax.experimental.pallas import tpu_sc as plsc

_B = 16384
_NA = 100
_OBS_W = 302
_L = 16
_NC = 2
_NS = 16
_NW = _NC * _NS
_RW = _B // _NW
_C = 64
_NCH = _RW // _C
_TPAD = _B + 128
_PEN = np.float32(3.0e38)
_GB = (0, 16, 32, 48, 64, 80, 84)
_BIG_IDX = np.int32(1 << 20)
_UNROLL = 2


@functools.partial(
    pl.kernel,
    out_type=jax.ShapeDtypeStruct((_B, _NA), jnp.float32),
    mesh=plsc.VectorSubcoreMesh(core_axis_name="c", subcore_axis_name="s"),
    compiler_params=pltpu.CompilerParams(needs_layout_passes=False),
    scratch_types=[
        pltpu.VMEM((_TPAD,), jnp.float32),
        pltpu.VMEM((_TPAD,), jnp.float32),
        pltpu.VMEM((_C, _OBS_W), jnp.float32),
        pltpu.VMEM((_C, _OBS_W), jnp.float32),
        pltpu.VMEM((_C, _NA), jnp.float32),
        pltpu.VMEM((_C, _NA), jnp.float32),
        pltpu.VMEM((_C, _NA), jnp.float32),
        pltpu.VMEM((_C, _NA), jnp.float32),
        pltpu.SemaphoreType.DMA,
        pltpu.SemaphoreType.DMA,
        pltpu.SemaphoreType.DMA,
        pltpu.SemaphoreType.DMA,
    ],
)
def _selector(obs_hbm, txp_hbm, typ_hbm, mask_hbm, out_hbm,
              tx_v, ty_v, obs_v0, obs_v1, mask_v0, mask_v1,
              out_v0, out_v1, sem_in0, sem_in1, sem_out0, sem_out1):
    wid = lax.axis_index("s") * _NC + lax.axis_index("c")
    base = wid * _RW
    pltpu.sync_copy(txp_hbm, tx_v)
    pltpu.sync_copy(typ_hbm, ty_v)

    obs_v = (obs_v0, obs_v1)
    mask_v = (mask_v0, mask_v1)
    out_v = (out_v0, out_v1)
    sem_in = (sem_in0, sem_in1)
    sem_out = (sem_out0, sem_out1)

    lane = lax.iota(jnp.int32, _L)
    one = jnp.float32(1.0)
    zero = jnp.float32(0.0)

    def start_in(k, b):
        r0 = base + k * _C
        c1 = pltpu.async_copy(obs_hbm.at[pl.ds(r0, _C)], obs_v[b], sem_in[b])
        c2 = pltpu.async_copy(mask_hbm.at[pl.ds(r0, _C)], mask_v[b], sem_in[b])
        return (c1, c2)

    pending_in = start_in(0, 0)
    pending_out = [None, None]
    for k in range(_NCH):
        b = k & 1
        for c in pending_in:
            c.wait()
        if k + 1 < _NCH:
            pending_in = start_in(k + 1, 1 - b)
        if pending_out[b] is not None:
            pending_out[b].wait()
        r0 = base + k * _C
        ob, mb, tb = obs_v[b], mask_v[b], out_v[b]

        @plsc.parallel_loop(0, _C, unroll=_UNROLL)
        def row(r):
            start = (jnp.int32(_NA) * (r0 + r)) & jnp.int32(_B - 1)
            rvec = jnp.full((_L,), r, jnp.int32)
            vmin = None
            varg = None
            for gb in _GB:
                ax = plsc.load_gather(ob, [rvec, lane * 3 + 3 * gb])
                ay = plsc.load_gather(ob, [rvec, lane * 3 + (3 * gb + 1)])
                tx = tx_v[pl.ds(start + gb, _L)]
                ty = ty_v[pl.ds(start + gb, _L)]
                mg = mb[r, pl.ds(gb, _L)]
                dx = ax - tx
                dy = ay - ty
                d = dx * dx + dy * dy + mg * _PEN
                ids = lane + gb
                if vmin is None:
                    vmin, varg = d, ids
                else:
                    lt = d < vmin
                    varg = jnp.where(lt, ids, varg)
                    vmin = jnp.where(lt, d, vmin)
            m = jnp.min(vmin)
            idx = jnp.min(jnp.where(vmin == m, varg, _BIG_IDX))
            for gb in _GB:
                oh = jnp.where(lane + gb == idx, one, zero).astype(jnp.float32)
                tb[r, pl.ds(gb, _L)] = oh

        pending_out[b] = pltpu.async_copy(
            tb, out_hbm.at[pl.ds(r0, _C)], sem_out[b])
    for b in (0, 1):
        if pending_out[b] is not None:
            pending_out[b].wait()


def kernel(input_obs, invalid_mask):
    tx = input_obs[:, _OBS_W - 2]
    ty = input_obs[:, _OBS_W - 1]
    txp = jnp.concatenate([tx, tx[: _TPAD - _B]])
    typ = jnp.concatenate([ty, ty[: _TPAD - _B]])
    return _selector(input_obs, txp, typ, invalid_mask)
